# baseline (speedup 1.0000x reference)
import jax
import jax.numpy as jnp
from jax import lax
from jax.experimental import pallas as pl
from jax.experimental.pallas import tpu as pltpu
from jax.experimental.pallas import tpu_sc as plsc

_E = 16
_LANES = 16
_R = 1024


def _logits_body(xa_ref, xb_ref, rc_ref, gw_ref, cw_ref, lt_ref, ctx_scr):
    b = pl.program_id(0)
    j = pl.program_id(1)

    @pl.when(jnp.logical_and(b == 0, j == 0))
    def _init():
        ctx_scr[...] = jax.lax.dot_general(
            rc_ref[...], cw_ref[...], (((1,), (1,)), ((), ())),
            preferred_element_type=jnp.float32)

    xb_full = jnp.concatenate([xa_ref[0], xb_ref[0]], axis=1)
    routing = xb_full + ctx_scr[pl.ds(b, 1), :]
    logits = jax.lax.dot_general(
        routing, gw_ref[...], (((1,), (1,)), ((), ())),
        preferred_element_type=jnp.float32)
    lt_ref[...] = logits.T


def _tc_logits(x, routing_context, gate_w, ctx_w):
    b, n, c = x.shape
    e = gate_w.shape[0]
    t = b * n
    ch = c // 2
    n_j = n // _R
    return pl.pallas_call(
        _logits_body,
        grid=(b, n_j),
        in_specs=[
            pl.BlockSpec((1, _R, ch), lambda bi, ji: (bi, ji, 0)),
            pl.BlockSpec((1, _R, ch), lambda bi, ji: (bi, ji, 1)),
            pl.BlockSpec((b, c), lambda bi, ji: (0, 0)),
            pl.BlockSpec((e, c), lambda bi, ji: (0, 0)),
            pl.BlockSpec((c, c), lambda bi, ji: (0, 0)),
        ],
        out_specs=pl.BlockSpec((e, _R), lambda bi, ji: (0, bi * n_j + ji)),
        out_shape=jax.ShapeDtypeStruct((e, t), jnp.float32),
        scratch_shapes=[pltpu.VMEM((b, c), jnp.float32)],
    )(x, x, routing_context, gate_w, ctx_w)


def _sc_router(lt, t):
    tok_per_w = t // 32

    def body(lt_hbm, idx_out, w_out, part_out,
             lv, i1buf, i2buf, w1buf, w2buf, imp_acc, load_acc, partbuf):
        wid = lax.axis_index("s") * 2 + lax.axis_index("c")
        base = wid * tok_per_w
        pltpu.sync_copy(lt_hbm.at[:, pl.ds(base, tok_per_w)], lv)

        lanes = lax.iota(jnp.int32, _LANES)
        zerosf = jnp.zeros((_LANES,), jnp.float32)
        for e in range(_E):
            imp_acc[e, :] = zerosf
            load_acc[e, :] = zerosf

        n_groups = tok_per_w // _LANES

        def group(g, carry):
            sl = pl.ds(g * _LANES, _LANES)
            vs = [lv[e, sl] for e in range(_E)]
            m1 = vs[0]
            i1 = jnp.zeros((_LANES,), jnp.int32)
            m2 = jnp.full((_LANES,), -3.0e38, jnp.float32)
            i2 = jnp.zeros((_LANES,), jnp.int32)
            for e in range(1, _E):
                v = vs[e]
                ev = jnp.full((_LANES,), e, jnp.int32)
                c1 = v > m1
                c2 = v > m2
                i2 = jnp.where(c1, i1, jnp.where(c2, ev, i2))
                m2 = jnp.where(c1, m1, jnp.where(c2, v, m2))
                i1 = jnp.where(c1, ev, i1)
                m1 = jnp.where(c1, v, m1)
            tw = jnp.exp(m2 - m1)
            rw = 1.0 / (1.0 + tw)
            i1buf[sl] = i1
            i2buf[sl] = i2
            w1buf[sl] = rw
            w2buf[sl] = tw * rw
            ps = [jnp.exp(v - m1) for v in vs]
            s = ps[0]
            for e in range(1, _E):
                s = s + ps[e]
            rs = 1.0 / s
            for e in range(_E):
                plsc.addupdate(imp_acc.at[e, :], ps[e] * rs)
                ev = jnp.full((_LANES,), e, jnp.int32)
                cnt = (jnp.where(i1 == ev, 1.0, 0.0) +
                       jnp.where(i2 == ev, 1.0, 0.0))
                plsc.addupdate(load_acc.at[e, :], cnt)
            return carry

        lax.fori_loop(0, n_groups, group, 0)

        imp_vec = zerosf
        load_vec = zerosf
        for e in range(_E):
            se = jnp.sum(imp_acc[e, :])
            imp_vec = jnp.where(lanes == e,
                                jnp.full((_LANES,), se, jnp.float32), imp_vec)
            le = jnp.sum(load_acc[e, :])
            load_vec = jnp.where(lanes == e,
                                 jnp.full((_LANES,), le, jnp.float32), load_vec)
        partbuf[0, :] = imp_vec
        partbuf[1, :] = load_vec

        pltpu.sync_copy(i1buf, idx_out.at[0, pl.ds(base, tok_per_w)])
        pltpu.sync_copy(i2buf, idx_out.at[1, pl.ds(base, tok_per_w)])
        pltpu.sync_copy(w1buf, w_out.at[0, pl.ds(base, tok_per_w)])
        pltpu.sync_copy(w2buf, w_out.at[1, pl.ds(base, tok_per_w)])
        pltpu.sync_copy(partbuf, part_out.at[wid])

    mesh = plsc.VectorSubcoreMesh(core_axis_name="c", subcore_axis_name="s")
    run = pl.kernel(
        body,
        compiler_params=pltpu.CompilerParams(needs_layout_passes=False),
        out_type=[
            jax.ShapeDtypeStruct((2, t), jnp.int32),
            jax.ShapeDtypeStruct((2, t), jnp.float32),
            jax.ShapeDtypeStruct((32, 2, _E), jnp.float32),
        ],
        mesh=mesh,
        scratch_types=[
            pltpu.VMEM((_E, tok_per_w), jnp.float32),
            pltpu.VMEM((tok_per_w,), jnp.int32),
            pltpu.VMEM((tok_per_w,), jnp.int32),
            pltpu.VMEM((tok_per_w,), jnp.float32),
            pltpu.VMEM((tok_per_w,), jnp.float32),
            pltpu.VMEM((_E, _LANES), jnp.float32),
            pltpu.VMEM((_E, _LANES), jnp.float32),
            pltpu.VMEM((2, _LANES), jnp.float32),
        ],
    )
    return run(lt)


def kernel(x, routing_context, gate_w, ctx_w):
    b, n, _ = x.shape
    e = gate_w.shape[0]
    t = b * n
    lt = _tc_logits(x, routing_context, gate_w, ctx_w)
    idx2, w2, parts = _sc_router(lt, t)
    top_idx = idx2.T
    top_w = w2.T
    imp = parts[:, 0, :].sum(axis=0) / float(t)
    load = parts[:, 1, :].sum(axis=0) / float(t)
    aux = float(e) * jnp.sum(imp * load)
    return (top_idx, top_w, aux)


# speedup vs baseline: 1.0077x; 1.0077x over previous
import jax
import jax.numpy as jnp
from jax import lax
from jax.experimental import pallas as pl
from jax.experimental.pallas import tpu as pltpu
from jax.experimental.pallas import tpu_sc as plsc

_E = 16
_LANES = 16
_R = 1024


def _logits_body(x_ref, rc_ref, gw_ref, cw_ref, lt_ref, ctx_scr):
    b = pl.program_id(0)
    j = pl.program_id(1)

    @pl.when(jnp.logical_and(b == 0, j == 0))
    def _init():
        ctx_scr[...] = jax.lax.dot_general(
            rc_ref[...], cw_ref[...], (((1,), (1,)), ((), ())),
            preferred_element_type=jnp.float32)

    routing = x_ref[0] + ctx_scr[pl.ds(b, 1), :]
    logits = jax.lax.dot_general(
        routing, gw_ref[...], (((1,), (1,)), ((), ())),
        preferred_element_type=jnp.float32)
    lt_ref[...] = logits.T


def _tc_logits(x, routing_context, gate_w, ctx_w):
    b, n, c = x.shape
    e = gate_w.shape[0]
    t = b * n
    n_j = n // _R
    return pl.pallas_call(
        _logits_body,
        grid=(b, n_j),
        in_specs=[
            pl.BlockSpec((1, _R, c), lambda bi, ji: (bi, ji, 0)),
            pl.BlockSpec((b, c), lambda bi, ji: (0, 0)),
            pl.BlockSpec((e, c), lambda bi, ji: (0, 0)),
            pl.BlockSpec((c, c), lambda bi, ji: (0, 0)),
        ],
        out_specs=pl.BlockSpec((e, _R), lambda bi, ji: (0, bi * n_j + ji)),
        out_shape=jax.ShapeDtypeStruct((e, t), jnp.float32),
        scratch_shapes=[pltpu.VMEM((b, c), jnp.float32)],
    )(x, routing_context, gate_w, ctx_w)


def _sc_router(lt, t):
    tok_per_w = t // 32

    def body(lt_hbm, idx_out, w_out, part_out,
             lv, i1buf, i2buf, w1buf, w2buf, imp_acc, load_acc, partbuf):
        wid = lax.axis_index("s") * 2 + lax.axis_index("c")
        base = wid * tok_per_w
        pltpu.sync_copy(lt_hbm.at[:, pl.ds(base, tok_per_w)], lv)

        lanes = lax.iota(jnp.int32, _LANES)
        zerosf = jnp.zeros((_LANES,), jnp.float32)
        zerosi = jnp.zeros((_LANES,), jnp.int32)
        onesi = jnp.ones((_LANES,), jnp.int32)
        for e in range(_E):
            imp_acc[e, :] = zerosf
        for k in range(4):
            load_acc[k, :] = zerosi

        n_groups = tok_per_w // _LANES

        def group(g, carry):
            sl = pl.ds(g * _LANES, _LANES)
            vs = [lv[e, sl] for e in range(_E)]
            m1 = vs[0]
            i1 = jnp.zeros((_LANES,), jnp.int32)
            m2 = jnp.full((_LANES,), -3.0e38, jnp.float32)
            i2 = jnp.zeros((_LANES,), jnp.int32)
            for e in range(1, _E):
                v = vs[e]
                ev = jnp.full((_LANES,), e, jnp.int32)
                c1 = v > m1
                c2 = v > m2
                i2 = jnp.where(c1, i1, jnp.where(c2, ev, i2))
                m2 = jnp.where(c1, m1, jnp.where(c2, v, m2))
                i1 = jnp.where(c1, ev, i1)
                m1 = jnp.where(c1, v, m1)
            tw = jnp.exp(m2 - m1)
            rw = 1.0 / (1.0 + tw)
            i1buf[sl] = i1
            i2buf[sl] = i2
            w1buf[sl] = rw
            w2buf[sl] = tw * rw
            ps = [jnp.exp(v - m1) for v in vs]
            s = ps[0]
            for e in range(1, _E):
                s = s + ps[e]
            rs = 1.0 / s
            for e in range(_E):
                plsc.addupdate(imp_acc.at[e, :], ps[e] * rs)
            b1 = onesi << ((i1 & 3) << 3)
            q1 = i1 >> 2
            b2 = onesi << ((i2 & 3) << 3)
            q2 = i2 >> 2
            for k in range(4):
                kv = jnp.full((_LANES,), k, jnp.int32)
                contrib = (jnp.where(q1 == kv, b1, zerosi) +
                           jnp.where(q2 == kv, b2, zerosi))
                plsc.addupdate(load_acc.at[k, :], contrib)
            return carry

        lax.fori_loop(0, n_groups, group, 0)

        imp_vec = zerosf
        load_vec = zerosf
        m255 = jnp.full((_LANES,), 255, jnp.int32)
        for e in range(_E):
            se = jnp.sum(imp_acc[e, :])
            imp_vec = jnp.where(lanes == e,
                                jnp.full((_LANES,), se, jnp.float32), imp_vec)
            cnts = ((load_acc[e // 4, :] >> (8 * (e % 4))) & m255)
            le = jnp.sum(cnts.astype(jnp.float32))
            load_vec = jnp.where(lanes == e,
                                 jnp.full((_LANES,), le, jnp.float32), load_vec)
        partbuf[0, :] = imp_vec
        partbuf[1, :] = load_vec

        pltpu.sync_copy(i1buf, idx_out.at[0, pl.ds(base, tok_per_w)])
        pltpu.sync_copy(i2buf, idx_out.at[1, pl.ds(base, tok_per_w)])
        pltpu.sync_copy(w1buf, w_out.at[0, pl.ds(base, tok_per_w)])
        pltpu.sync_copy(w2buf, w_out.at[1, pl.ds(base, tok_per_w)])
        pltpu.sync_copy(partbuf, part_out.at[wid])

    mesh = plsc.VectorSubcoreMesh(core_axis_name="c", subcore_axis_name="s")
    run = pl.kernel(
        body,
        compiler_params=pltpu.CompilerParams(needs_layout_passes=False),
        out_type=[
            jax.ShapeDtypeStruct((2, t), jnp.int32),
            jax.ShapeDtypeStruct((2, t), jnp.float32),
            jax.ShapeDtypeStruct((32, 2, _E), jnp.float32),
        ],
        mesh=mesh,
        scratch_types=[
            pltpu.VMEM((_E, tok_per_w), jnp.float32),
            pltpu.VMEM((tok_per_w,), jnp.int32),
            pltpu.VMEM((tok_per_w,), jnp.int32),
            pltpu.VMEM((tok_per_w,), jnp.float32),
            pltpu.VMEM((tok_per_w,), jnp.float32),
            pltpu.VMEM((_E, _LANES), jnp.float32),
            pltpu.VMEM((4, _LANES), jnp.int32),
            pltpu.VMEM((2, _LANES), jnp.float32),
        ],
    )
    return run(lt)


def kernel(x, routing_context, gate_w, ctx_w):
    b, n, _ = x.shape
    e = gate_w.shape[0]
    t = b * n
    lt = _tc_logits(x, routing_context, gate_w, ctx_w)
    idx2, w2, parts = _sc_router(lt, t)
    top_idx = idx2.T
    top_w = w2.T
    imp = parts[:, 0, :].sum(axis=0) / float(t)
    load = parts[:, 1, :].sum(axis=0) / float(t)
    aux = float(e) * jnp.sum(imp * load)
    return (top_idx, top_w, aux)
